# Initial kernel scaffold; baseline (speedup 1.0000x reference)
import jax
import jax.numpy as jnp
from jax.experimental import pallas as pl


def kernel(x, em_weight, of_weight):
    raise NotImplementedError("write your pallas kernel here")


# speedup vs baseline: 3.8899x; 3.8899x over previous
import functools

import jax
import jax.numpy as jnp
from jax import lax
from jax.experimental import pallas as pl
from jax.experimental.pallas import tpu as pltpu
from jax.experimental.pallas import tpu_sc as plsc

B, L = 4096, 200
GLOVE, FEAT = 128, 64
D = GLOVE + FEAT
VOCAB = 100000
BL = B * L

NC, NS = 2, 16
NW = NC * NS
GL = 128
G_TOT = BL // GL
G_PER_W = G_TOT // NW
CHUNK_G = 2
N_CHUNK = G_PER_W // CHUNK_G

_mesh = plsc.VectorSubcoreMesh(
    core_axis_name="c", subcore_axis_name="s", num_cores=NC, num_subcores=NS
)


@functools.partial(
    pl.kernel,
    out_type=jax.ShapeDtypeStruct((G_TOT, GL, D), jnp.float32),
    mesh=_mesh,
    scratch_types=[
        pltpu.VMEM((CHUNK_G, GL), jnp.int32),
        pltpu.VMEM((CHUNK_G, GL, D), jnp.float32),
        pltpu.VMEM((CHUNK_G, GL, GLOVE), jnp.float32),
        pltpu.SemaphoreType.DMA,
    ],
)
def _sc_gather(x_hbm, em_hbm, ofp_hbm, out_hbm, idx_v, comb_v, of_v, sem):
    wid = lax.axis_index("s") * NC + lax.axis_index("c")
    g_base = wid * G_PER_W

    @pl.loop(0, N_CHUNK)
    def _chunk(ci):
        g0 = g_base + ci * CHUNK_G
        pltpu.sync_copy(x_hbm.at[pl.ds(g0, CHUNK_G)], idx_v)
        descs = []
        for g in range(CHUNK_G):
            descs.append(
                pltpu.async_copy(
                    em_hbm.at[idx_v.at[g]], comb_v.at[g, :, pl.ds(0, GLOVE)], sem
                )
            )
            descs.append(pltpu.async_copy(ofp_hbm.at[idx_v.at[g]], of_v.at[g], sem))
        for d in descs:
            d.wait()
        for g in range(CHUNK_G):

            @pl.loop(0, GL)
            def _row(r, g=g):
                for c in range(FEAT // 16):
                    comb_v[g, r, pl.ds(GLOVE + c * 16, 16)] = of_v[
                        g, r, pl.ds(c * 16, 16)
                    ]

        pltpu.sync_copy(comb_v, out_hbm.at[pl.ds(g0, CHUNK_G)])


def _pad_body(of_ref, o_ref):
    o_ref[:, 0:FEAT] = of_ref[...]
    o_ref[:, FEAT:GLOVE] = jnp.zeros((of_ref.shape[0], GLOVE - FEAT), jnp.float32)


_PAD_ROWS = 2000
_pad_call = pl.pallas_call(
    _pad_body,
    grid=(VOCAB // _PAD_ROWS,),
    in_specs=[pl.BlockSpec((_PAD_ROWS, FEAT), lambda i: (i, 0))],
    out_specs=pl.BlockSpec((_PAD_ROWS, GLOVE), lambda i: (i, 0)),
    out_shape=jax.ShapeDtypeStruct((VOCAB, GLOVE), jnp.float32),
)


def _mask_body(x_ref, o_ref):
    o_ref[...] = x_ref[...] > 0


_mask_call = pl.pallas_call(
    _mask_body,
    out_shape=jax.ShapeDtypeStruct((B, L), jnp.bool_),
)


def kernel(x, em_weight, of_weight):
    x2 = x.reshape(G_TOT, GL)
    of_p = _pad_call(of_weight)
    out = _sc_gather(x2, em_weight, of_p)
    mask = _mask_call(x)
    return out.reshape(B, L, D), mask
